# baseline (speedup 1.0000x reference)
import jax
import jax.numpy as jnp
from jax.experimental import pallas as pl
from jax.experimental.pallas import tpu as pltpu

_EPS = 1e-6


def _elu1(x):
    return jnp.where(x > 0, x + 1.0, jnp.exp(x))


def kernel(queries, keys, values, solved_sample):
    del solved_sample
    N, L, H, D = queries.shape
    F = H * D
    grid = (N,)

    q2 = queries.reshape(N, L, F).astype(jnp.bfloat16)
    k2 = keys.reshape(N, L, F).astype(jnp.bfloat16)
    v2 = values.reshape(N, L, F).astype(jnp.bfloat16)

    def kv_kernel(k_ref, v_ref, m_ref):
        k = _elu1(k_ref[0].astype(jnp.float32)).astype(jnp.bfloat16)
        v = v_ref[0]
        rhs = jnp.concatenate([v, jnp.ones_like(v)], axis=1)
        kv = jax.lax.dot_general(
            k, rhs, (((0,), (0,)), ((), ())),
            preferred_element_type=jnp.float32)
        i = jax.lax.broadcasted_iota(jnp.int32, (F, 2 * F), 0)
        j = jax.lax.broadcasted_iota(jnp.int32, (F, 2 * F), 1)
        mask = (i // D) == ((j % F) // D)
        m_ref[0] = jnp.where(mask, kv, 0.0)

    m = pl.pallas_call(
        kv_kernel,
        grid=grid,
        in_specs=[
            pl.BlockSpec((1, L, F), lambda n: (n, 0, 0)),
            pl.BlockSpec((1, L, F), lambda n: (n, 0, 0)),
        ],
        out_specs=pl.BlockSpec((1, F, 2 * F), lambda n: (n, 0, 0)),
        out_shape=jax.ShapeDtypeStruct((N, F, 2 * F), jnp.float32),
        compiler_params=pltpu.CompilerParams(
            dimension_semantics=("parallel",)),
    )(k2, v2)

    def out_kernel(m_ref, q_ref, o_ref):
        q = _elu1(q_ref[0].astype(jnp.float32)).astype(jnp.bfloat16)
        r = jax.lax.dot_general(
            q, m_ref[0].astype(jnp.bfloat16), (((1,), (0,)), ((), ())),
            preferred_element_type=jnp.float32)
        o_ref[0] = (r[:, :F] / (r[:, F:] + _EPS)).astype(jnp.bfloat16)

    out = pl.pallas_call(
        out_kernel,
        grid=grid,
        in_specs=[
            pl.BlockSpec((1, F, 2 * F), lambda n: (n, 0, 0)),
            pl.BlockSpec((1, L, F), lambda n: (n, 0, 0)),
        ],
        out_specs=pl.BlockSpec((1, L, F), lambda n: (n, 0, 0)),
        out_shape=jax.ShapeDtypeStruct((N, L, F), jnp.bfloat16),
        compiler_params=pltpu.CompilerParams(
            dimension_semantics=("parallel",)),
    )(m, q2)

    return out.astype(jnp.float32).reshape(N, L, H, D)


# speedup vs baseline: 1.1210x; 1.1210x over previous
import jax
import jax.numpy as jnp
from jax.experimental import pallas as pl
from jax.experimental.pallas import tpu as pltpu

_EPS = 1e-6


def _elu1(x):
    return jnp.where(x > 0, x + 1.0, jnp.exp(x))


def kernel(queries, keys, values, solved_sample):
    del solved_sample
    N, L, H, D = queries.shape
    F = H * D
    grid = (N,)

    q2 = queries.reshape(N, L, F)
    k2 = keys.reshape(N, L, F)
    v2 = values.reshape(N, L, F)

    def kv_kernel(k_ref, v_ref, m_ref):
        k = _elu1(k_ref[0]).astype(jnp.bfloat16)
        v = v_ref[0].astype(jnp.bfloat16)
        rhs = jnp.concatenate([v, jnp.ones_like(v)], axis=1)
        kv = jax.lax.dot_general(
            k, rhs, (((0,), (0,)), ((), ())),
            preferred_element_type=jnp.float32)
        i = jax.lax.broadcasted_iota(jnp.int32, (F, 2 * F), 0)
        j = jax.lax.broadcasted_iota(jnp.int32, (F, 2 * F), 1)
        mask = (i // D) == ((j % F) // D)
        m_ref[0] = jnp.where(mask, kv, 0.0)

    m = pl.pallas_call(
        kv_kernel,
        grid=grid,
        in_specs=[
            pl.BlockSpec((1, L, F), lambda n: (n, 0, 0)),
            pl.BlockSpec((1, L, F), lambda n: (n, 0, 0)),
        ],
        out_specs=pl.BlockSpec((1, F, 2 * F), lambda n: (n, 0, 0)),
        out_shape=jax.ShapeDtypeStruct((N, F, 2 * F), jnp.float32),
        compiler_params=pltpu.CompilerParams(
            dimension_semantics=("parallel",)),
    )(k2, v2)

    def out_kernel(m_ref, q_ref, o_ref):
        q = _elu1(q_ref[0]).astype(jnp.bfloat16)
        r = jax.lax.dot_general(
            q, m_ref[0].astype(jnp.bfloat16), (((1,), (0,)), ((), ())),
            preferred_element_type=jnp.float32)
        o_ref[0] = (r[:, :F] / (r[:, F:] + _EPS)).astype(jnp.bfloat16)

    out = pl.pallas_call(
        out_kernel,
        grid=grid,
        in_specs=[
            pl.BlockSpec((1, F, 2 * F), lambda n: (n, 0, 0)),
            pl.BlockSpec((1, L, F), lambda n: (n, 0, 0)),
        ],
        out_specs=pl.BlockSpec((1, L, F), lambda n: (n, 0, 0)),
        out_shape=jax.ShapeDtypeStruct((N, L, F), jnp.bfloat16),
        compiler_params=pltpu.CompilerParams(
            dimension_semantics=("parallel",)),
    )(m, q2)

    return out.astype(jnp.float32).reshape(N, L, H, D)


# speedup vs baseline: 1.1300x; 1.0080x over previous
import jax
import jax.numpy as jnp
from jax.experimental import pallas as pl
from jax.experimental.pallas import tpu as pltpu

_EPS = 1e-6


def _elu1(x):
    return jnp.where(x > 0, x + 1.0, jnp.exp(x))


def kernel(queries, keys, values, solved_sample):
    del solved_sample
    N, L, H, D = queries.shape
    F = H * D
    grid = (N,)

    q2 = queries.reshape(N, L, F)
    k2 = keys.reshape(N, L, F)
    v2 = values.reshape(N, L, F)

    def kv_kernel(k_ref, v_ref, m_ref):
        k = _elu1(k_ref[0]).astype(jnp.bfloat16)
        v = v_ref[0].astype(jnp.bfloat16)
        rhs = jnp.concatenate([v, jnp.ones_like(v)], axis=1)
        kv = jax.lax.dot_general(
            k, rhs, (((0,), (0,)), ((), ())),
            preferred_element_type=jnp.float32)
        i = jax.lax.broadcasted_iota(jnp.int32, (F, 2 * F), 0)
        j = jax.lax.broadcasted_iota(jnp.int32, (F, 2 * F), 1)
        mask = (i // D) == ((j % F) // D)
        m_ref[0] = jnp.where(mask, kv, 0.0).astype(jnp.bfloat16)

    m = pl.pallas_call(
        kv_kernel,
        grid=grid,
        in_specs=[
            pl.BlockSpec((1, L, F), lambda n: (n, 0, 0)),
            pl.BlockSpec((1, L, F), lambda n: (n, 0, 0)),
        ],
        out_specs=pl.BlockSpec((1, F, 2 * F), lambda n: (n, 0, 0)),
        out_shape=jax.ShapeDtypeStruct((N, F, 2 * F), jnp.bfloat16),
        compiler_params=pltpu.CompilerParams(
            dimension_semantics=("parallel",)),
    )(k2, v2)

    def out_kernel(m_ref, q_ref, o_ref):
        q = _elu1(q_ref[0]).astype(jnp.bfloat16)
        r = jax.lax.dot_general(
            q, m_ref[0], (((1,), (0,)), ((), ())),
            preferred_element_type=jnp.float32)
        o_ref[0] = (r[:, :F] / (r[:, F:] + _EPS)).astype(jnp.bfloat16)

    out = pl.pallas_call(
        out_kernel,
        grid=grid,
        in_specs=[
            pl.BlockSpec((1, F, 2 * F), lambda n: (n, 0, 0)),
            pl.BlockSpec((1, L, F), lambda n: (n, 0, 0)),
        ],
        out_specs=pl.BlockSpec((1, L, F), lambda n: (n, 0, 0)),
        out_shape=jax.ShapeDtypeStruct((N, L, F), jnp.bfloat16),
        compiler_params=pltpu.CompilerParams(
            dimension_semantics=("parallel",)),
    )(m, q2)

    return out.astype(jnp.float32).reshape(N, L, H, D)
